# baseline (speedup 1.0000x reference)
import functools

import jax
import jax.numpy as jnp
from jax import lax
from jax.experimental import pallas as pl
from jax.experimental.pallas import tpu as pltpu
from jax.experimental.pallas import tpu_sc as plsc

N = 50000
HALF = 25000
PCORE = 25088
STRIPE = PCORE // 16
CH = 128
DH = 64
BLK = 2000
N_LEVELS = 8


def _sc_msg_body(agg_hbm, srcs_hbm, dstl_hbm, starts_hbm, ends_hbm, zeros_hbm,
                 msg_hbm, sidx, didx, rows, sv, ev, shared, sem):
    cid = lax.axis_index("c")
    sid = lax.axis_index("s")
    w = cid * 16 + sid

    pltpu.sync_copy(starts_hbm.at[pl.ds(w * 16, 16)], sv)
    pltpu.sync_copy(ends_hbm.at[pl.ds(w * 16, 16)], ev)
    s0 = sv[...][0]
    e0 = ev[...][0]
    start8 = (s0 // 8) * 8
    nch = jnp.maximum((e0 - start8 + CH - 1) // CH, 0)

    pltpu.sync_copy(zeros_hbm, shared.at[pl.ds(sid * STRIPE, STRIPE)])
    plsc.subcore_barrier()

    trash = HALF + sid

    def body(k, carry):
        base = start8 + k * CH
        pltpu.sync_copy(srcs_hbm.at[pl.ds(base, CH)], sidx)
        pltpu.sync_copy(dstl_hbm.at[pl.ds(base, CH)], didx)
        for g in range(CH // 16):
            pos = base + g * 16 + lax.iota(jnp.int32, 16)
            valid = (pos >= s0) & (pos < e0)
            sg = sidx[pl.ds(g * 16, 16)]
            dg = didx[pl.ds(g * 16, 16)]
            sidx[pl.ds(g * 16, 16)] = jnp.where(valid, sg, 0)
            didx[pl.ds(g * 16, 16)] = jnp.where(valid, dg, trash)
        pltpu.async_copy(agg_hbm.at[sidx], rows, sem).wait()
        pltpu.sync_copy(rows, shared.at[didx], add=True)
        return carry

    lax.fori_loop(0, nch, body, 0)
    plsc.subcore_barrier()

    pltpu.sync_copy(shared.at[pl.ds(sid * STRIPE, STRIPE)],
                    msg_hbm.at[pl.ds(cid * PCORE + sid * STRIPE, STRIPE)])


@functools.cache
def _sc_msg():
    return pl.kernel(
        _sc_msg_body,
        mesh=plsc.VectorSubcoreMesh(core_axis_name="c", subcore_axis_name="s"),
        compiler_params=pltpu.CompilerParams(use_tc_tiling_on_sc=False),
        out_type=jax.ShapeDtypeStruct((2 * PCORE, DH), jnp.float32),
        scratch_types=[
            pltpu.VMEM((CH,), jnp.int32),
            pltpu.VMEM((CH,), jnp.int32),
            pltpu.VMEM((CH, DH), jnp.float32),
            pltpu.VMEM((16,), jnp.int32),
            pltpu.VMEM((16,), jnp.int32),
            pltpu.VMEM_SHARED((PCORE, DH), jnp.float32),
            pltpu.SemaphoreType.DMA,
        ],
    )


def _init_body(x_ref, h0_ref, wihx_ref, bih_ref, wagg_ref, bagg_ref,
               gx_ref, h_ref, agg_ref):
    gx_ref[...] = jnp.dot(x_ref[...], wihx_ref[...],
                          preferred_element_type=jnp.float32) + bih_ref[...]
    h0 = h0_ref[...]
    h_ref[...] = jnp.broadcast_to(h0, h_ref.shape)
    agg0 = jnp.dot(h0, wagg_ref[...],
                   preferred_element_type=jnp.float32) + bagg_ref[...]
    agg_ref[...] = jnp.broadcast_to(agg0, agg_ref.shape)


def _gru_body(h_ref, msg_ref, gx_ref, mask_ref, wihm_ref, whh_ref, bhh_ref,
              wagg_ref, bagg_ref, hout_ref, aggout_ref):
    h = h_ref[...]
    gh = jnp.dot(h, whh_ref[...], preferred_element_type=jnp.float32) + bhh_ref[...]
    gi = jnp.dot(msg_ref[...], wihm_ref[...],
                 preferred_element_type=jnp.float32) + gx_ref[...]
    r = jax.nn.sigmoid(gi[:, :DH] + gh[:, :DH])
    z = jax.nn.sigmoid(gi[:, DH:2 * DH] + gh[:, DH:2 * DH])
    n = jnp.tanh(gi[:, 2 * DH:] + r * gh[:, 2 * DH:])
    hnew = (1.0 - z) * n + z * h
    m = mask_ref[...] > 0.5
    hout = jnp.where(m, hnew, h)
    hout_ref[...] = hout
    aggout_ref[...] = jnp.dot(hout, wagg_ref[...],
                              preferred_element_type=jnp.float32) + bagg_ref[...]


def _mlp_body(h_ref, w1_ref, b1_ref, w2_ref, b2_ref, o_ref):
    hid = jnp.maximum(
        jnp.dot(h_ref[...], w1_ref[...], preferred_element_type=jnp.float32)
        + b1_ref[...], 0.0)
    o_ref[...] = jnp.dot(hid, w2_ref[...],
                         preferred_element_type=jnp.float32) + b2_ref[...]


def _full(shape):
    return pl.BlockSpec(shape, lambda i: (0, 0))


def _blk(shape):
    return pl.BlockSpec(shape, lambda i: (i, 0))


def kernel(x, edge_index, forward_level, forward_index, backward_level,
           backward_index, W_emd, b_emd, W_agg, b_agg, W_ih, b_ih, W_hh, b_hh,
           W1, b1, W2, b2):
    E = edge_index.shape[1]
    grid = (N // BLK,)

    e_src = edge_index[0].astype(jnp.int32)
    e_dst = edge_index[1].astype(jnp.int32)
    lvl_d = forward_level[e_dst].astype(jnp.int32)
    halfb = (e_dst >= HALF).astype(jnp.int32)
    key = lvl_d * 2 + halfb
    oh = (key[:, None] == jnp.arange(16, dtype=jnp.int32)[None, :]).astype(jnp.int32)
    csum = jnp.cumsum(oh, axis=0)
    counts = csum[-1]
    offs = jnp.concatenate([jnp.zeros((1,), jnp.int32),
                            jnp.cumsum(counts)[:-1].astype(jnp.int32)])
    rank = jnp.take_along_axis(csum, key[:, None], axis=1)[:, 0] - 1
    position = offs[key] + rank
    inv = jnp.zeros((E,), jnp.int32).at[position].set(
        jnp.arange(E, dtype=jnp.int32), unique_indices=True,
        mode="promise_in_bounds")
    srcs = e_src[inv]
    dsts = e_dst[inv]
    dstl = dsts - HALF * (dsts >= HALF).astype(jnp.int32)
    ss = jnp.concatenate([offs, jnp.array([E], jnp.int32)])
    pad = jnp.zeros((2 * CH,), jnp.int32)
    srcs_p = jnp.concatenate([srcs, pad])
    dstl_p = jnp.concatenate([dstl, pad])

    lane16 = jnp.arange(16, dtype=jnp.int32)

    def worker_bounds(a, b):
        wsz = -((a - b) // 16)
        s = jnp.minimum(a + lane16 * wsz, b)
        e = jnp.minimum(s + wsz, b)
        return s, e

    starts_by_l, ends_by_l = [], []
    for l in range(1, N_LEVELS):
        s0, e0 = worker_bounds(ss[2 * l], ss[2 * l + 1])
        s1, e1 = worker_bounds(ss[2 * l + 1], ss[2 * l + 2])
        sw = jnp.concatenate([s0, s1]).astype(jnp.int32)
        ew = jnp.concatenate([e0, e1]).astype(jnp.int32)
        starts_by_l.append(jnp.repeat(sw, 16))
        ends_by_l.append(jnp.repeat(ew, 16))

    zeros_stripe = jnp.zeros((STRIPE, DH), jnp.float32)

    h0row = (W_emd[:, 0] + b_emd).reshape(1, DH)
    wihm_t = W_ih[:, :DH].T
    wihx_t = W_ih[:, DH:].T
    whh_t = W_hh.T
    wagg_t = W_agg.T
    bih_r = b_ih.reshape(1, 3 * DH)
    bhh_r = b_hh.reshape(1, 3 * DH)
    bagg_r = b_agg.reshape(1, DH)
    w1_t = W1.T
    w2_t = W2.T
    b1_r = b1.reshape(1, -1)
    b2_r = b2.reshape(1, -1)

    gx, h, agg = pl.pallas_call(
        _init_body,
        grid=grid,
        in_specs=[_blk((BLK, x.shape[1])), _full((1, DH)),
                  _full(wihx_t.shape), _full((1, 3 * DH)),
                  _full((DH, DH)), _full((1, DH))],
        out_specs=[_blk((BLK, 3 * DH)), _blk((BLK, DH)), _blk((BLK, DH))],
        out_shape=[jax.ShapeDtypeStruct((N, 3 * DH), jnp.float32),
                   jax.ShapeDtypeStruct((N, DH), jnp.float32),
                   jax.ShapeDtypeStruct((N, DH), jnp.float32)],
    )(x, h0row, wihx_t, bih_r, wagg_t, bagg_r)

    for l in range(1, N_LEVELS):
        msg_raw = _sc_msg()(agg, srcs_p, dstl_p, starts_by_l[l - 1],
                            ends_by_l[l - 1], zeros_stripe)
        msgf = jnp.concatenate(
            [msg_raw[:HALF], msg_raw[PCORE:PCORE + HALF]], axis=0)
        maskf = (forward_level == l).astype(jnp.float32).reshape(N, 1)
        h, agg = pl.pallas_call(
            _gru_body,
            grid=grid,
            in_specs=[_blk((BLK, DH)), _blk((BLK, DH)), _blk((BLK, 3 * DH)),
                      _blk((BLK, 1)), _full((DH, 3 * DH)), _full((DH, 3 * DH)),
                      _full((1, 3 * DH)), _full((DH, DH)), _full((1, DH))],
            out_specs=[_blk((BLK, DH)), _blk((BLK, DH))],
            out_shape=[jax.ShapeDtypeStruct((N, DH), jnp.float32),
                       jax.ShapeDtypeStruct((N, DH), jnp.float32)],
        )(h, msgf, gx, maskf, wihm_t, whh_t, bhh_r, wagg_t, bagg_r)

    out = pl.pallas_call(
        _mlp_body,
        grid=grid,
        in_specs=[_blk((BLK, DH)), _full(w1_t.shape), _full((1, w1_t.shape[1])),
                  _full(w2_t.shape), _full((1, 1))],
        out_specs=_blk((BLK, 1)),
        out_shape=jax.ShapeDtypeStruct((N, 1), jnp.float32),
    )(h, w1_t, b1_r, w2_t, b2_r)
    return out


# speedup vs baseline: 1.2969x; 1.2969x over previous
import functools

import jax
import jax.numpy as jnp
from jax import lax
from jax.experimental import pallas as pl
from jax.experimental.pallas import tpu as pltpu
from jax.experimental.pallas import tpu_sc as plsc

N = 50000
HALF = 25000
PCORE = 25088
STRIPE = PCORE // 16
CH = 128
DH = 64
BLK = 2000
N_LEVELS = 8


def _sc_msg_body(agg_hbm, srcs_hbm, dstl_hbm, starts_hbm, ends_hbm, zeros_hbm,
                 msg_hbm, sidx, didx, rows, sv, ev, shared, sem):
    cid = lax.axis_index("c")
    sid = lax.axis_index("s")
    w = cid * 16 + sid

    pltpu.sync_copy(starts_hbm.at[pl.ds(w * 16, 16)], sv)
    pltpu.sync_copy(ends_hbm.at[pl.ds(w * 16, 16)], ev)
    s0 = sv[...][0]
    e0 = ev[...][0]
    start8 = (s0 // 8) * 8
    nch = jnp.maximum((e0 - start8 + CH - 1) // CH, 0)

    pltpu.sync_copy(zeros_hbm, shared.at[pl.ds(sid * STRIPE, STRIPE)])
    plsc.subcore_barrier()

    trash = HALF + sid

    def body(k, carry):
        base = start8 + k * CH
        pltpu.sync_copy(srcs_hbm.at[pl.ds(base, CH)], sidx)
        pltpu.sync_copy(dstl_hbm.at[pl.ds(base, CH)], didx)
        for g in range(CH // 16):
            pos = base + g * 16 + lax.iota(jnp.int32, 16)
            valid = (pos >= s0) & (pos < e0)
            sg = sidx[pl.ds(g * 16, 16)]
            dg = didx[pl.ds(g * 16, 16)]
            sidx[pl.ds(g * 16, 16)] = jnp.where(valid, sg, 0)
            didx[pl.ds(g * 16, 16)] = jnp.where(valid, dg, trash)
        pltpu.async_copy(agg_hbm.at[sidx], rows, sem).wait()
        pltpu.sync_copy(rows, shared.at[didx], add=True)
        return carry

    lax.fori_loop(0, nch, body, 0)
    plsc.subcore_barrier()

    pltpu.sync_copy(shared.at[pl.ds(sid * STRIPE, STRIPE)],
                    msg_hbm.at[pl.ds(cid * PCORE + sid * STRIPE, STRIPE)])


@functools.cache
def _sc_msg():
    return pl.kernel(
        _sc_msg_body,
        mesh=plsc.VectorSubcoreMesh(core_axis_name="c", subcore_axis_name="s"),
        compiler_params=pltpu.CompilerParams(use_tc_tiling_on_sc=False),
        out_type=jax.ShapeDtypeStruct((2 * PCORE, DH), jnp.float32),
        scratch_types=[
            pltpu.VMEM((CH,), jnp.int32),
            pltpu.VMEM((CH,), jnp.int32),
            pltpu.VMEM((CH, DH), jnp.float32),
            pltpu.VMEM((16,), jnp.int32),
            pltpu.VMEM((16,), jnp.int32),
            pltpu.VMEM_SHARED((PCORE, DH), jnp.float32),
            pltpu.SemaphoreType.DMA,
        ],
    )


def _init_body(x_ref, h0_ref, wihx_ref, bih_ref, wagg_ref, bagg_ref,
               gx_ref, h_ref, agg_ref):
    gx_ref[...] = jnp.dot(x_ref[...], wihx_ref[...],
                          preferred_element_type=jnp.float32) + bih_ref[...]
    h0 = h0_ref[...]
    h_ref[...] = jnp.broadcast_to(h0, h_ref.shape)
    agg0 = jnp.dot(h0, wagg_ref[...],
                   preferred_element_type=jnp.float32) + bagg_ref[...]
    agg_ref[...] = jnp.broadcast_to(agg0, agg_ref.shape)


def _gru_body(h_ref, msg_ref, gx_ref, mask_ref, wihm_ref, whh_ref, bhh_ref,
              wagg_ref, bagg_ref, hout_ref, aggout_ref):
    h = h_ref[...]
    gh = jnp.dot(h, whh_ref[...], preferred_element_type=jnp.float32) + bhh_ref[...]
    gi = jnp.dot(msg_ref[...], wihm_ref[...],
                 preferred_element_type=jnp.float32) + gx_ref[...]
    r = jax.nn.sigmoid(gi[:, :DH] + gh[:, :DH])
    z = jax.nn.sigmoid(gi[:, DH:2 * DH] + gh[:, DH:2 * DH])
    n = jnp.tanh(gi[:, 2 * DH:] + r * gh[:, 2 * DH:])
    hnew = (1.0 - z) * n + z * h
    m = mask_ref[...] > 0.5
    hout = jnp.where(m, hnew, h)
    hout_ref[...] = hout
    aggout_ref[...] = jnp.dot(hout, wagg_ref[...],
                              preferred_element_type=jnp.float32) + bagg_ref[...]


def _mlp_body(h_ref, w1_ref, b1_ref, w2_ref, b2_ref, o_ref):
    hid = jnp.maximum(
        jnp.dot(h_ref[...], w1_ref[...], preferred_element_type=jnp.float32)
        + b1_ref[...], 0.0)
    o_ref[...] = jnp.dot(hid, w2_ref[...],
                         preferred_element_type=jnp.float32) + b2_ref[...]


def _full(shape):
    return pl.BlockSpec(shape, lambda i: (0, 0))


def _blk(shape):
    return pl.BlockSpec(shape, lambda i: (i, 0))


def kernel(x, edge_index, forward_level, forward_index, backward_level,
           backward_index, W_emd, b_emd, W_agg, b_agg, W_ih, b_ih, W_hh, b_hh,
           W1, b1, W2, b2):
    E = edge_index.shape[1]
    grid = (N // BLK,)

    e_src = edge_index[0].astype(jnp.int32)
    e_dst = edge_index[1].astype(jnp.int32)
    lvl_d = forward_level[e_dst].astype(jnp.int32)
    halfb = (e_dst >= HALF).astype(jnp.int32)
    key = lvl_d * 2 + halfb
    dstl0 = e_dst - HALF * halfb
    key_s, srcs, dstl = jax.lax.sort((key, e_src, dstl0), num_keys=1)
    ss = jnp.searchsorted(key_s, jnp.arange(2 * N_LEVELS + 1, dtype=jnp.int32))
    pad = jnp.zeros((2 * CH,), jnp.int32)
    srcs_p = jnp.concatenate([srcs, pad])
    dstl_p = jnp.concatenate([dstl, pad])

    lane16 = jnp.arange(16, dtype=jnp.int32)

    def worker_bounds(a, b):
        wsz = -((a - b) // 16)
        s = jnp.minimum(a + lane16 * wsz, b)
        e = jnp.minimum(s + wsz, b)
        return s, e

    starts_by_l, ends_by_l = [], []
    for l in range(1, N_LEVELS):
        s0, e0 = worker_bounds(ss[2 * l], ss[2 * l + 1])
        s1, e1 = worker_bounds(ss[2 * l + 1], ss[2 * l + 2])
        sw = jnp.concatenate([s0, s1]).astype(jnp.int32)
        ew = jnp.concatenate([e0, e1]).astype(jnp.int32)
        starts_by_l.append(jnp.repeat(sw, 16))
        ends_by_l.append(jnp.repeat(ew, 16))

    zeros_stripe = jnp.zeros((STRIPE, DH), jnp.float32)

    h0row = (W_emd[:, 0] + b_emd).reshape(1, DH)
    wihm_t = W_ih[:, :DH].T
    wihx_t = W_ih[:, DH:].T
    whh_t = W_hh.T
    wagg_t = W_agg.T
    bih_r = b_ih.reshape(1, 3 * DH)
    bhh_r = b_hh.reshape(1, 3 * DH)
    bagg_r = b_agg.reshape(1, DH)
    w1_t = W1.T
    w2_t = W2.T
    b1_r = b1.reshape(1, -1)
    b2_r = b2.reshape(1, -1)

    gx, h, agg = pl.pallas_call(
        _init_body,
        grid=grid,
        in_specs=[_blk((BLK, x.shape[1])), _full((1, DH)),
                  _full(wihx_t.shape), _full((1, 3 * DH)),
                  _full((DH, DH)), _full((1, DH))],
        out_specs=[_blk((BLK, 3 * DH)), _blk((BLK, DH)), _blk((BLK, DH))],
        out_shape=[jax.ShapeDtypeStruct((N, 3 * DH), jnp.float32),
                   jax.ShapeDtypeStruct((N, DH), jnp.float32),
                   jax.ShapeDtypeStruct((N, DH), jnp.float32)],
    )(x, h0row, wihx_t, bih_r, wagg_t, bagg_r)

    for l in range(1, N_LEVELS):
        msg_raw = _sc_msg()(agg, srcs_p, dstl_p, starts_by_l[l - 1],
                            ends_by_l[l - 1], zeros_stripe)
        msgf = jnp.concatenate(
            [msg_raw[:HALF], msg_raw[PCORE:PCORE + HALF]], axis=0)
        maskf = (forward_level == l).astype(jnp.float32).reshape(N, 1)
        h, agg = pl.pallas_call(
            _gru_body,
            grid=grid,
            in_specs=[_blk((BLK, DH)), _blk((BLK, DH)), _blk((BLK, 3 * DH)),
                      _blk((BLK, 1)), _full((DH, 3 * DH)), _full((DH, 3 * DH)),
                      _full((1, 3 * DH)), _full((DH, DH)), _full((1, DH))],
            out_specs=[_blk((BLK, DH)), _blk((BLK, DH))],
            out_shape=[jax.ShapeDtypeStruct((N, DH), jnp.float32),
                       jax.ShapeDtypeStruct((N, DH), jnp.float32)],
        )(h, msgf, gx, maskf, wihm_t, whh_t, bhh_r, wagg_t, bagg_r)

    out = pl.pallas_call(
        _mlp_body,
        grid=grid,
        in_specs=[_blk((BLK, DH)), _full(w1_t.shape), _full((1, w1_t.shape[1])),
                  _full(w2_t.shape), _full((1, 1))],
        out_specs=_blk((BLK, 1)),
        out_shape=jax.ShapeDtypeStruct((N, 1), jnp.float32),
    )(h, w1_t, b1_r, w2_t, b2_r)
    return out
